# Initial kernel scaffold; baseline (speedup 1.0000x reference)
import jax
import jax.numpy as jnp
from jax.experimental import pallas as pl


def kernel(x, edge_index, Wk0, bk0, Ws0, bs0, Wk1, bk1, Ws1, bs1, Wk2, bk2, Ws2, bs2, Wm, bm):
    raise NotImplementedError("write your pallas kernel here")


# speedup vs baseline: 8.3213x; 8.3213x over previous
import functools

import jax
import jax.numpy as jnp
from jax import lax
from jax.experimental import pallas as pl
from jax.experimental.pallas import tpu as pltpu
from jax.experimental.pallas import tpu_sc as plsc

NCORES = 2
NSUB = 16
NW = NCORES * NSUB
CH = 128
ZR = 104


def _make_s_kernel(n, e, w, gather):
    tot_chunks = e // CH
    base_chunks = tot_chunks // NW
    rem = tot_chunks - base_chunks * NW
    exp_rows = (n // NSUB) // 8 * 8
    tail_rows = n - exp_rows * NSUB
    assert exp_rows % ZR == 0 and tail_rows % 8 == 0 and tail_rows <= CH

    mesh = plsc.VectorSubcoreMesh(core_axis_name="c", subcore_axis_name="s")

    scratch = [
        pltpu.VMEM((CH,), jnp.int32),
        pltpu.VMEM((CH,), jnp.int32),
        pltpu.VMEM((CH, w), jnp.float32),
        pltpu.VMEM_SHARED((n, w), jnp.float32),
        pltpu.SemaphoreType.DMA,
    ]

    def body(*refs):
        if gather:
            g_hbm, ei_hbm, out_hbm, colv, rowv, rows, acc, sem = refs
        else:
            ei_hbm, out_hbm, colv, rowv, rows, acc, sem = refs
            g_hbm = None
        cid = lax.axis_index("c")
        sid = lax.axis_index("s")
        wid = sid * NCORES + cid

        zero16 = jnp.zeros((16,), jnp.float32)

        def zrow(r, carry):
            for cc in range(w // 16):
                rows[r, pl.ds(cc * 16, 16)] = zero16
            return carry

        lax.fori_loop(0, CH, zrow, 0)
        row0 = sid * exp_rows
        for jz in range(exp_rows // ZR):
            pltpu.sync_copy(rows.at[pl.ds(0, ZR)],
                            acc.at[pl.ds(row0 + jz * ZR, ZR)])

        @pl.when(sid == NSUB - 1)
        def _():
            pltpu.sync_copy(rows.at[pl.ds(0, tail_rows)],
                            acc.at[pl.ds(exp_rows * NSUB, tail_rows)])

        plsc.subcore_barrier()

        if not gather:
            one16 = jnp.ones((16,), jnp.float32)

            def orow(r, carry):
                for cc in range(w // 16):
                    rows[r, pl.ds(cc * 16, 16)] = one16
                return carry

            lax.fori_loop(0, CH, orow, 0)

        start = base_chunks * wid + jnp.minimum(wid, rem)
        nch = base_chunks + (wid < rem).astype(jnp.int32)

        def do_chunk(j, carry):
            eb = (start + j) * CH
            pltpu.sync_copy(ei_hbm.at[0, pl.ds(eb, CH)], rowv)
            if gather:
                pltpu.sync_copy(ei_hbm.at[1, pl.ds(eb, CH)], colv)
                pltpu.async_copy(g_hbm.at[colv], rows, sem).wait()
            pltpu.sync_copy(rows, acc.at[rowv], add=True)
            return carry

        lax.fori_loop(0, nch, do_chunk, 0)

        plsc.subcore_barrier()
        pltpu.sync_copy(acc.at[pl.ds(row0, exp_rows)],
                        out_hbm.at[cid, pl.ds(row0, exp_rows)])

        @pl.when(sid == NSUB - 1)
        def _():
            pltpu.sync_copy(acc.at[pl.ds(exp_rows * NSUB, tail_rows)],
                            out_hbm.at[cid, pl.ds(exp_rows * NSUB, tail_rows)])

    return pl.kernel(
        body,
        out_type=jax.ShapeDtypeStruct((NCORES, n, w), jnp.float32),
        mesh=mesh,
        scratch_types=scratch,
        compiler_params=pltpu.CompilerParams(use_tc_tiling_on_sc=False),
    )


def kernel(x, edge_index, Wk0, bk0, Ws0, bs0, Wk1, bk1, Ws1, bs1,
           Wk2, bk2, Ws2, bs2, Wm, bm):
    n, d = x.shape
    e = edge_index.shape[1]
    nh = Wk0.shape[2]
    w48 = 48
    w16 = 16

    s48 = _make_s_kernel(n, e, w48, gather=True)
    s16 = _make_s_kernel(n, e, w16, gather=True)
    degk = _make_s_kernel(n, e, w16, gather=False)

    degp = degk(edge_index)
    deg = degp[0, :, 0] + degp[1, :, 0]
    dis = jnp.where(deg > 0, lax.rsqrt(jnp.maximum(deg, 1e-12)), 0.0)
    disc = dis[:, None]

    def S48(v):
        g = jnp.zeros((n, w48), jnp.float32).at[:, :nh].set(v)
        p = s48(g, edge_index)
        return (p[0] + p[1])[:, :nh]

    def cheb6(h, Wk, bk):
        y = jnp.einsum("nd,kdf->knf", h, Wk)
        b2 = jnp.zeros_like(y[0])
        b1 = y[5]
        for k in range(4, 0, -1):
            b0 = y[k] - 2.0 * disc * S48(disc * b1) - b2
            b1, b2 = b0, b1
        return y[0] + bk - disc * S48(disc * b1) - b2

    h = x
    for (Wk, bk, Ws, bs) in ((Wk0, bk0, Ws0, bs0), (Wk1, bk1, Ws1, bs1),
                             (Wk2, bk2, Ws2, bs2)):
        h = jax.nn.relu(cheb6(h, Wk, bk)) + (h @ Ws[0] + bs)

    hc = jnp.concatenate([h, x], axis=1)
    v = hc @ Wm[1]
    g = jnp.zeros((n, w16), jnp.float32).at[:, :v.shape[1]].set(disc * v)
    p = s16(g, edge_index)
    sv = (p[0] + p[1])[:, :v.shape[1]]
    return hc @ Wm[0] - disc * sv + bm


# speedup vs baseline: 23.2225x; 2.7907x over previous
import functools

import jax
import jax.numpy as jnp
from jax import lax
from jax.experimental import pallas as pl
from jax.experimental.pallas import tpu as pltpu
from jax.experimental.pallas import tpu_sc as plsc

NCORES = 2
NSUB = 16
NW = NCORES * NSUB
CH = 128
ZR = 104
NBUF = 6


def _make_s_kernel(n, e, w, gather):
    tot_chunks = e // CH
    base_chunks = tot_chunks // NW
    rem = tot_chunks - base_chunks * NW
    maxc = base_chunks + (1 if rem else 0)
    exp_rows = (n // NSUB) // 8 * 8
    tail_rows = n - exp_rows * NSUB
    assert exp_rows % ZR == 0 and tail_rows % 8 == 0 and tail_rows <= CH

    mesh = plsc.VectorSubcoreMesh(core_axis_name="c", subcore_axis_name="s")

    scratch = [
        pltpu.VMEM((maxc, CH), jnp.int32),
        pltpu.VMEM((maxc, CH), jnp.int32),
        pltpu.VMEM((NBUF, CH, w), jnp.float32),
        pltpu.VMEM_SHARED((n, w), jnp.float32),
        pltpu.SemaphoreType.DMA((NBUF,)),
        pltpu.SemaphoreType.DMA((NBUF,)),
    ]

    def body(*refs):
        if gather:
            g_hbm, ei_hbm, out_hbm, colbig, rowbig, rows, acc, gsem, ssem = refs
        else:
            ei_hbm, out_hbm, colbig, rowbig, rows, acc, gsem, ssem = refs
            g_hbm = None
        cid = lax.axis_index("c")
        sid = lax.axis_index("s")
        wid = sid * NCORES + cid

        zero16 = jnp.zeros((16,), jnp.float32)

        def zrow(r, carry):
            for cc in range(w // 16):
                rows[0, r, pl.ds(cc * 16, 16)] = zero16
            return carry

        lax.fori_loop(0, CH, zrow, 0)
        row0 = sid * exp_rows
        for jz in range(exp_rows // ZR):
            pltpu.sync_copy(rows.at[0, pl.ds(0, ZR)],
                            acc.at[pl.ds(row0 + jz * ZR, ZR)])

        @pl.when(sid == NSUB - 1)
        def _():
            pltpu.sync_copy(rows.at[0, pl.ds(0, tail_rows)],
                            acc.at[pl.ds(exp_rows * NSUB, tail_rows)])

        start = base_chunks * wid + jnp.minimum(wid, rem)
        nch = base_chunks + (wid < rem).astype(jnp.int32)
        pltpu.sync_copy(ei_hbm.at[0, pl.ds(start, base_chunks)],
                        rowbig.at[pl.ds(0, base_chunks)])
        if gather:
            pltpu.sync_copy(ei_hbm.at[1, pl.ds(start, base_chunks)],
                            colbig.at[pl.ds(0, base_chunks)])
        if rem:
            @pl.when(wid < rem)
            def _():
                pltpu.sync_copy(ei_hbm.at[0, pl.ds(start + base_chunks, 1)],
                                rowbig.at[pl.ds(base_chunks, 1)])
                if gather:
                    pltpu.sync_copy(ei_hbm.at[1, pl.ds(start + base_chunks, 1)],
                                    colbig.at[pl.ds(base_chunks, 1)])

        plsc.subcore_barrier()

        if gather:
            for b in range(NBUF - 1):
                pltpu.async_copy(g_hbm.at[colbig.at[b]], rows.at[b],
                                 gsem.at[b])

            def do_chunk(j, carry):
                b = lax.rem(j, NBUF)
                pltpu.make_async_copy(g_hbm.at[colbig.at[j]], rows.at[b],
                                      gsem.at[b]).wait()
                pltpu.async_copy(rows.at[b], acc.at[rowbig.at[j]],
                                 ssem.at[b], add=True)
                jn = j + NBUF - 1
                bn = lax.rem(jn, NBUF)

                @pl.when(j > 0)
                def _():
                    pltpu.make_async_copy(
                        rows.at[bn], acc.at[rowbig.at[j - 1]],
                        ssem.at[bn]).wait()

                @pl.when(jn < nch)
                def _():
                    pltpu.async_copy(g_hbm.at[colbig.at[jn]], rows.at[bn],
                                     gsem.at[bn])
                return carry

            lax.fori_loop(0, nch, do_chunk, 0)
            bl = lax.rem(nch - 1, NBUF)
            pltpu.make_async_copy(rows.at[bl], acc.at[rowbig.at[nch - 1]],
                                  ssem.at[bl]).wait()
        else:
            one16 = jnp.ones((16,), jnp.float32)

            def orow(r, carry):
                for cc in range(w // 16):
                    rows[0, r, pl.ds(cc * 16, 16)] = one16
                return carry

            lax.fori_loop(0, CH, orow, 0)

            def do_chunk(j, carry):
                pltpu.sync_copy(rows.at[0], acc.at[rowbig.at[j]], add=True)
                return carry

            lax.fori_loop(0, nch, do_chunk, 0)

        plsc.subcore_barrier()
        pltpu.sync_copy(acc.at[pl.ds(row0, exp_rows)],
                        out_hbm.at[cid, pl.ds(row0, exp_rows)])

        @pl.when(sid == NSUB - 1)
        def _():
            pltpu.sync_copy(acc.at[pl.ds(exp_rows * NSUB, tail_rows)],
                            out_hbm.at[cid, pl.ds(exp_rows * NSUB, tail_rows)])

    return pl.kernel(
        body,
        out_type=jax.ShapeDtypeStruct((NCORES, n, w), jnp.float32),
        mesh=mesh,
        scratch_types=scratch,
        compiler_params=pltpu.CompilerParams(use_tc_tiling_on_sc=False),
    )


def kernel(x, edge_index, Wk0, bk0, Ws0, bs0, Wk1, bk1, Ws1, bs1,
           Wk2, bk2, Ws2, bs2, Wm, bm):
    n, d = x.shape
    e = edge_index.shape[1]
    nh = Wk0.shape[2]
    w48 = 48
    w16 = 16

    s48 = _make_s_kernel(n, e, w48, gather=True)
    s16 = _make_s_kernel(n, e, w16, gather=True)
    degk = _make_s_kernel(n, e, w16, gather=False)

    ei3 = edge_index.reshape(2, e // CH, CH)
    degp = degk(ei3)
    deg = degp[0, :, 0] + degp[1, :, 0]
    dis = jnp.where(deg > 0, lax.rsqrt(jnp.maximum(deg, 1e-12)), 0.0)
    disc = dis[:, None]

    def S48(v):
        g = jnp.zeros((n, w48), jnp.float32).at[:, :nh].set(v)
        p = s48(g, ei3)
        return (p[0] + p[1])[:, :nh]

    def cheb6(h, Wk, bk):
        y = jnp.einsum("nd,kdf->knf", h, Wk)
        b2 = jnp.zeros_like(y[0])
        b1 = y[5]
        for k in range(4, 0, -1):
            b0 = y[k] - 2.0 * disc * S48(disc * b1) - b2
            b1, b2 = b0, b1
        return y[0] + bk - disc * S48(disc * b1) - b2

    h = x
    for (Wk, bk, Ws, bs) in ((Wk0, bk0, Ws0, bs0), (Wk1, bk1, Ws1, bs1),
                             (Wk2, bk2, Ws2, bs2)):
        h = jax.nn.relu(cheb6(h, Wk, bk)) + (h @ Ws[0] + bs)

    hc = jnp.concatenate([h, x], axis=1)
    v = hc @ Wm[1]
    g = jnp.zeros((n, w16), jnp.float32).at[:, :v.shape[1]].set(disc * v)
    p = s16(g, ei3)
    sv = (p[0] + p[1])[:, :v.shape[1]]
    return hc @ Wm[0] - disc * sv + bm


# speedup vs baseline: 23.9008x; 1.0292x over previous
import functools

import jax
import jax.numpy as jnp
from jax import lax
from jax.experimental import pallas as pl
from jax.experimental.pallas import tpu as pltpu
from jax.experimental.pallas import tpu_sc as plsc

NCORES = 2
NSUB = 16
NW = NCORES * NSUB
CH = 128
ZR = 104
NBUF = 6


def _make_s_kernel(n, e, w, gather):
    tot_chunks = e // CH
    base_chunks = tot_chunks // NW
    rem = tot_chunks - base_chunks * NW
    maxc = base_chunks + (1 if rem else 0)
    exp_rows = (n // NSUB) // 8 * 8
    tail_rows = n - exp_rows * NSUB
    assert exp_rows % ZR == 0 and tail_rows % 8 == 0 and tail_rows <= CH

    mesh = plsc.VectorSubcoreMesh(core_axis_name="c", subcore_axis_name="s")

    scratch = [
        pltpu.VMEM((maxc, CH), jnp.int32),
        pltpu.VMEM((maxc, CH), jnp.int32),
        pltpu.VMEM((NBUF, CH, w), jnp.float32),
        pltpu.VMEM_SHARED((n, w), jnp.float32),
        pltpu.SemaphoreType.DMA((NBUF,)),
        pltpu.SemaphoreType.DMA((NBUF,)),
    ]

    def body(*refs):
        if gather:
            z_hbm, g_hbm, ei_hbm, out_hbm, colbig, rowbig, rows, acc, gsem, ssem = refs
        else:
            z_hbm, ei_hbm, out_hbm, colbig, rowbig, rows, acc, gsem, ssem = refs
            g_hbm = None
        cid = lax.axis_index("c")
        sid = lax.axis_index("s")
        wid = sid * NCORES + cid

        row0 = sid * exp_rows
        pltpu.sync_copy(z_hbm.at[pl.ds(row0, exp_rows)],
                        acc.at[pl.ds(row0, exp_rows)])

        @pl.when(sid == NSUB - 1)
        def _():
            pltpu.sync_copy(z_hbm.at[pl.ds(exp_rows * NSUB, tail_rows)],
                            acc.at[pl.ds(exp_rows * NSUB, tail_rows)])

        start = base_chunks * wid + jnp.minimum(wid, rem)
        nch = base_chunks + (wid < rem).astype(jnp.int32)
        pltpu.sync_copy(ei_hbm.at[0, pl.ds(start, base_chunks)],
                        rowbig.at[pl.ds(0, base_chunks)])
        if gather:
            pltpu.sync_copy(ei_hbm.at[1, pl.ds(start, base_chunks)],
                            colbig.at[pl.ds(0, base_chunks)])
        if rem:
            @pl.when(wid < rem)
            def _():
                pltpu.sync_copy(ei_hbm.at[0, pl.ds(start + base_chunks, 1)],
                                rowbig.at[pl.ds(base_chunks, 1)])
                if gather:
                    pltpu.sync_copy(ei_hbm.at[1, pl.ds(start + base_chunks, 1)],
                                    colbig.at[pl.ds(base_chunks, 1)])

        plsc.subcore_barrier()

        if gather:
            for b in range(NBUF - 1):
                pltpu.async_copy(g_hbm.at[colbig.at[b]], rows.at[b],
                                 gsem.at[b])

            def do_chunk(j, carry):
                b = lax.rem(j, NBUF)
                pltpu.make_async_copy(g_hbm.at[colbig.at[j]], rows.at[b],
                                      gsem.at[b]).wait()
                pltpu.async_copy(rows.at[b], acc.at[rowbig.at[j]],
                                 ssem.at[b], add=True)
                jn = j + NBUF - 1
                bn = lax.rem(jn, NBUF)

                @pl.when(j > 0)
                def _():
                    pltpu.make_async_copy(
                        rows.at[bn], acc.at[rowbig.at[j - 1]],
                        ssem.at[bn]).wait()

                @pl.when(jn < nch)
                def _():
                    pltpu.async_copy(g_hbm.at[colbig.at[jn]], rows.at[bn],
                                     gsem.at[bn])
                return carry

            lax.fori_loop(0, nch, do_chunk, 0)
            bl = lax.rem(nch - 1, NBUF)
            pltpu.make_async_copy(rows.at[bl], acc.at[rowbig.at[nch - 1]],
                                  ssem.at[bl]).wait()
        else:
            one16 = jnp.ones((16,), jnp.float32)

            def orow(r, carry):
                for cc in range(w // 16):
                    rows[0, r, pl.ds(cc * 16, 16)] = one16
                return carry

            lax.fori_loop(0, CH, orow, 0)

            def do_chunk(j, carry):
                pltpu.sync_copy(rows.at[0], acc.at[rowbig.at[j]], add=True)
                return carry

            lax.fori_loop(0, nch, do_chunk, 0)

        plsc.subcore_barrier()
        pltpu.sync_copy(acc.at[pl.ds(row0, exp_rows)],
                        out_hbm.at[cid, pl.ds(row0, exp_rows)])

        @pl.when(sid == NSUB - 1)
        def _():
            pltpu.sync_copy(acc.at[pl.ds(exp_rows * NSUB, tail_rows)],
                            out_hbm.at[cid, pl.ds(exp_rows * NSUB, tail_rows)])

    return pl.kernel(
        body,
        out_type=jax.ShapeDtypeStruct((NCORES, n, w), jnp.float32),
        mesh=mesh,
        scratch_types=scratch,
        compiler_params=pltpu.CompilerParams(use_tc_tiling_on_sc=False),
    )


def _padw(Wt, dinp, doutp):
    K, din, dout = Wt.shape
    return jnp.zeros((K, dinp, doutp), jnp.float32).at[:, :din, :dout].set(Wt)


def _padv(b, doutp):
    return jnp.zeros((doutp,), jnp.float32).at[:b.shape[0]].set(b)


def kernel(x, edge_index, Wk0, bk0, Ws0, bs0, Wk1, bk1, Ws1, bs1,
           Wk2, bk2, Ws2, bs2, Wm, bm):
    n, d = x.shape
    e = edge_index.shape[1]
    nh = Wk0.shape[2]
    wp = 40
    wf = 16

    sP = _make_s_kernel(n, e, wp, gather=True)
    sF = _make_s_kernel(n, e, wf, gather=True)
    degk = _make_s_kernel(n, e, wf, gather=False)

    zp = jnp.zeros((n, wp), jnp.float32)
    zf = jnp.zeros((n, wf), jnp.float32)
    ei3 = edge_index.reshape(2, e // CH, CH)

    degp = degk(zf, ei3)
    deg = degp[0, :, 0] + degp[1, :, 0]
    dis = jnp.where(deg > 0, lax.rsqrt(jnp.maximum(deg, 1e-12)), 0.0)
    disc = dis[:, None]
    tdisc2 = 2.0 * disc * disc

    def S(g):
        p = sP(zp, g, ei3)
        return p[0] + p[1]

    h = x
    for (Wk, bk, Ws, bs) in ((Wk0, bk0, Ws0, bs0), (Wk1, bk1, Ws1, bs1),
                             (Wk2, bk2, Ws2, bs2)):
        dinp = h.shape[1]
        y = jnp.einsum("nd,kdf->knf", h, _padw(Wk, dinp, wp))
        g5 = disc * y[5]
        S5 = S(g5)
        g4 = disc * y[4] - tdisc2 * S5
        S4 = S(g4)
        g3 = disc * y[3] - tdisc2 * S4 - g5
        S3 = S(g3)
        g2 = disc * y[2] - tdisc2 * S3 - g4
        S2 = S(g2)
        g1 = disc * y[1] - tdisc2 * S2 - g3
        S1 = S(g1)
        b4 = y[4] - 2.0 * disc * S5
        b2 = y[2] - 2.0 * disc * S3 - b4
        out6 = y[0] + _padv(bk, wp) - disc * S1 - b2
        h = jax.nn.relu(out6) + (h @ _padw(Ws, dinp, wp)[0] + _padv(bs, wp))

    wm1h = _padw(Wm[1][:nh][None], h.shape[1], wf)[0]
    wm1x = _padw(Wm[1][nh:][None], d, wf)[0]
    v = h @ wm1h + x @ wm1x
    p = sF(zf, disc * v, ei3)
    sv = (p[0] + p[1])[:, :Wm.shape[2]]
    wm0h = _padw(Wm[0][:nh][None], h.shape[1], Wm.shape[2])[0]
    return h @ wm0h + x @ Wm[0][nh:] - disc * sv + bm
